# baseline (speedup 1.0000x reference)
import functools

import jax
import jax.numpy as jnp
from jax import lax
from jax.experimental import pallas as pl
from jax.experimental.pallas import tpu as pltpu
from jax.experimental.pallas import tpu_sc as plsc

BATCH = 16384
D = 64

_NC, _NS = 2, 16
_NW = _NC * _NS
_BPW = BATCH // _NW


def _sc_gather(user_table, movie_table, user_ids, movie_ids):
    mesh = plsc.VectorSubcoreMesh(core_axis_name="c", subcore_axis_name="s")

    @functools.partial(
        pl.kernel,
        mesh=mesh,
        out_type=[
            jax.ShapeDtypeStruct((BATCH, D), jnp.float32),
            jax.ShapeDtypeStruct((BATCH, D), jnp.float32),
        ],
        scratch_types=[
            pltpu.VMEM((_BPW,), jnp.int32),
            pltpu.VMEM((_BPW, D), jnp.float32),
            pltpu.VMEM((_BPW,), jnp.int32),
            pltpu.VMEM((_BPW, D), jnp.float32),
            pltpu.SemaphoreType.DMA,
            pltpu.SemaphoreType.DMA,
        ],
        compiler_params=pltpu.CompilerParams(use_tc_tiling_on_sc=False),
    )
    def gather_kernel(utab, mtab, uids, mids, uout, mout,
                      uidx_v, urows_v, midx_v, mrows_v, usem, msem):
        wid = lax.axis_index("s") * _NC + lax.axis_index("c")
        base = wid * _BPW
        pltpu.sync_copy(uids.at[pl.ds(base, _BPW)], uidx_v)
        pltpu.sync_copy(mids.at[pl.ds(base, _BPW)], midx_v)
        cu = pltpu.async_copy(utab.at[uidx_v], urows_v, usem)
        cm = pltpu.async_copy(mtab.at[midx_v], mrows_v, msem)
        cu.wait()
        cm.wait()
        pltpu.sync_copy(urows_v, uout.at[pl.ds(base, _BPW)])
        pltpu.sync_copy(mrows_v, mout.at[pl.ds(base, _BPW)])

    return gather_kernel(user_table, movie_table, user_ids, movie_ids)


def _mlp_body(uf_ref, mf_ref, w1u_ref, w1m_ref, b1_ref, g1_ref, be1_ref,
              w2_ref, b2_ref, g2_ref, be2_ref, w3_ref, b3_ref, out_ref):
    h = jnp.dot(uf_ref[...], w1u_ref[...], preferred_element_type=jnp.float32)
    h = h + jnp.dot(mf_ref[...], w1m_ref[...], preferred_element_type=jnp.float32)
    h = h + b1_ref[...]
    mu = jnp.mean(h, axis=-1, keepdims=True)
    var = jnp.mean((h - mu) ** 2, axis=-1, keepdims=True)
    h = (h - mu) * lax.rsqrt(var + 1e-5) * g1_ref[...] + be1_ref[...]
    h = jnp.maximum(h, 0.0)

    h = jnp.dot(h, w2_ref[...], preferred_element_type=jnp.float32) + b2_ref[...]
    mu = jnp.mean(h, axis=-1, keepdims=True)
    var = jnp.mean((h - mu) ** 2, axis=-1, keepdims=True)
    h = (h - mu) * lax.rsqrt(var + 1e-5) * g2_ref[...] + be2_ref[...]
    h = jnp.maximum(h, 0.0)

    o = jnp.sum(h * w3_ref[...], axis=-1, keepdims=True) + b3_ref[...]
    out_ref[...] = 5.5 / (1.0 + jnp.exp(-o))


def _tc_mlp(uf, mf, W1, b1, g1, be1, W2, b2, g2, be2, W3, b3):
    H1 = W1.shape[1]
    H2 = W2.shape[1]
    BB = 2048
    grid = (BATCH // BB,)

    def xmap(i):
        return (i, 0)

    def wmap(i):
        return (0, 0)

    return pl.pallas_call(
        _mlp_body,
        grid=grid,
        in_specs=[
            pl.BlockSpec((BB, D), xmap),
            pl.BlockSpec((BB, D), xmap),
            pl.BlockSpec((D, H1), wmap),
            pl.BlockSpec((D, H1), wmap),
            pl.BlockSpec((1, H1), wmap),
            pl.BlockSpec((1, H1), wmap),
            pl.BlockSpec((1, H1), wmap),
            pl.BlockSpec((H1, H2), wmap),
            pl.BlockSpec((1, H2), wmap),
            pl.BlockSpec((1, H2), wmap),
            pl.BlockSpec((1, H2), wmap),
            pl.BlockSpec((1, H2), wmap),
            pl.BlockSpec((1, 1), wmap),
        ],
        out_specs=pl.BlockSpec((BB, 1), xmap),
        out_shape=jax.ShapeDtypeStruct((BATCH, 1), jnp.float32),
    )(
        uf, mf,
        W1[:D], W1[D:],
        b1.reshape(1, H1), g1.reshape(1, H1), be1.reshape(1, H1),
        W2,
        b2.reshape(1, H2), g2.reshape(1, H2), be2.reshape(1, H2),
        W3.reshape(1, H2),
        b3.reshape(1, 1),
    )


def kernel(user_ids, movie_ids, user_table, movie_table,
           W1, b1, g1, be1, W2, b2, g2, be2, W3, b3):
    uf, mf = _sc_gather(user_table, movie_table,
                        user_ids.astype(jnp.int32), movie_ids.astype(jnp.int32))
    return _tc_mlp(uf, mf, W1, b1, g1, be1, W2, b2, g2, be2, W3, b3)
